# baseline (speedup 1.0000x reference)
import functools

import jax
import jax.numpy as jnp
import numpy as np
from jax import lax
from jax.experimental import pallas as pl
from jax.experimental.pallas import tpu as pltpu
from jax.experimental.pallas import tpu_sc as plsc

_INT_MIN = np.int32(-(2**31))


def _score_select_body(keep_n, xt_ref, w1t_ref, b1_ref, w2_ref, b2_ref,
                       a_ref, t_ref, s_ref):
    bb, d, n = xt_ref.shape

    @pl.when(pl.program_id(0) == 0)
    def _():
        r = lax.broadcasted_iota(jnp.int32, (n, n), 0)
        c = lax.broadcasted_iota(jnp.int32, (n, n), 1)
        t_ref[...] = (r <= c).astype(jnp.float32)

    for r in range(bb):
        h = jnp.dot(w1t_ref[...], xt_ref[r], preferred_element_type=jnp.float32)
        h = h + b1_ref[...]
        h = 0.5 * h * (1.0 + lax.erf(h * np.float32(0.7071067811865476)))
        s_ref[pl.ds(r, 1), :] = jnp.dot(w2_ref[...], h,
                                        preferred_element_type=jnp.float32)
    s = s_ref[...] + b2_ref[0, 0]

    bits = lax.bitcast_convert_type(s, jnp.int32)
    key = jnp.where(bits >= 0, bits, bits ^ np.int32(0x7FFFFFFF))
    cu = jnp.zeros((bb, 1), jnp.int32)
    for bit in range(31, -1, -1):
        bitval = _INT_MIN if bit == 31 else np.int32(1 << bit)
        cand = cu | bitval
        cand_s = cand ^ _INT_MIN
        cnt = jnp.sum((key >= cand_s).astype(jnp.int32), axis=1, keepdims=True)
        cu = jnp.where(cnt >= keep_n, cand, cu)
    t_s = cu ^ _INT_MIN

    gt = key > t_s
    eq = key == t_s
    c_gt = jnp.sum(gt.astype(jnp.int32), axis=1, keepdims=True)
    need = (keep_n - c_gt).astype(jnp.float32)
    eqf = eq.astype(jnp.float32)
    eqcum = jnp.dot(eqf, t_ref[...], preferred_element_type=jnp.float32)
    tie = eq & ((eqcum - eqf) < need)
    keepm = gt | tie
    keepf = keepm.astype(jnp.float32)
    dcum = jnp.dot(keepf, t_ref[...], preferred_element_type=jnp.float32)
    dest = dcum.astype(jnp.int32) - 1
    a_ref[...] = jnp.where(keepm, dest, np.int32(-1))


def _sc_gather_body(n, d, keep, rw,
                    xt_hbm, a_hbm, out_hbm, xbuf, dest_v, idx_v, ob0, sem0):
    cid = lax.axis_index("c")
    sid = lax.axis_index("s")
    wid = sid * 2 + cid
    iota = lax.iota(jnp.int32, 16)
    zeros16 = jnp.zeros((16,), jnp.int32)
    nch = 12

    def row_body(r, carry):
        b = wid * rw + r
        pltpu.sync_copy(xt_hbm.at[pl.ds(b * d, d)], xbuf)
        pltpu.sync_copy(a_hbm.at[b], dest_v)

        zbase = (keep // 16) * 16
        def zbody(c, carry2):
            idx_v[pl.ds(zbase + c * 16, 16)] = zeros16
            return carry2

        lax.fori_loop(0, (nch * 64 - zbase) // 16, zbody, 0)

        def cbody(c, carry2):
            dchunk = dest_v[pl.ds(c * 16, 16)]
            msk = (dchunk >= 0) & (dchunk < nch * 64)
            dsafe = jnp.where(msk, dchunk, 0)
            plsc.store_scatter(idx_v, [dsafe], iota + c * 16, mask=msk)
            return carry2

        lax.fori_loop(0, n // 16, cbody, 0)

        def chunk(c, carry2):
            def gbody(g, carry3):
                src16 = idx_v[pl.ds(c * 64 + g * 16, 16)]
                o16 = g * 16 + iota
                for dd in range(d):
                    dsplat = jnp.full((16,), dd, jnp.int32)
                    v = plsc.load_gather(xbuf, [dsplat, src16])
                    plsc.store_scatter(ob0, [o16, dsplat], v)
                return carry3

            lax.fori_loop(0, 4, gbody, 0)
            pltpu.async_copy(ob0, out_hbm.at[b, c], sem0).wait()
            return carry2

        lax.fori_loop(0, nch, chunk, 0)
        return carry

    lax.fori_loop(0, rw, row_body, 0)


def kernel(x, W1, b1, W2, b2):
    B, N, D = x.shape
    H = W1.shape[1]
    KEEP = (N * 7) // 10

    xt = jnp.transpose(x, (0, 2, 1))
    w1t = jnp.transpose(W1)

    BB = 8
    A = pl.pallas_call(
        functools.partial(_score_select_body, KEEP),
        grid=(B // BB,),
        in_specs=[
            pl.BlockSpec((BB, D, N), lambda i: (i, 0, 0)),
            pl.BlockSpec((H, D), lambda i: (0, 0)),
            pl.BlockSpec((H, N), lambda i: (0, 0)),
            pl.BlockSpec((1, H), lambda i: (0, 0)),
            pl.BlockSpec((1, 1), lambda i: (0, 0)),
        ],
        out_specs=pl.BlockSpec((BB, N), lambda i: (i, 0)),
        out_shape=jax.ShapeDtypeStruct((B, N), jnp.int32),
        scratch_shapes=[pltpu.VMEM((N, N), jnp.float32),
                        pltpu.VMEM((BB, N), jnp.float32)],
    )(xt, w1t, jnp.broadcast_to(b1.reshape(H, 1), (H, N)), W2.reshape(1, H),
      b2.reshape(1, 1))

    info = plsc.get_sparse_core_info()
    assert info.num_cores == 2 and info.num_subcores == 16
    RW = B // 32

    xt2d = xt.reshape(B * D, N)
    sc_fn = functools.partial(_sc_gather_body, N, D, KEEP, RW)
    out4 = pl.kernel(
        sc_fn,
        mesh=plsc.VectorSubcoreMesh(core_axis_name="c", subcore_axis_name="s"),
        compiler_params=pltpu.CompilerParams(needs_layout_passes=False),
        out_type=jax.ShapeDtypeStruct((B, 12, 64, D), jnp.float32),
        scratch_types=[
            pltpu.VMEM((D, N), jnp.float32),
            pltpu.VMEM((N,), jnp.int32),
            pltpu.VMEM((6 * 128,), jnp.int32),
            pltpu.VMEM((64, D), jnp.float32),
            pltpu.SemaphoreType.DMA,
        ],
    )(xt2d, A)

    return out4.reshape(B, 12 * 64, D)[:, :KEEP, :]
